# baseline (speedup 1.0000x reference)
import jax
import jax.numpy as jnp
from jax.experimental import pallas as pl


_BLOCK_ROWS = 512


def _add_kernel(x_ref, pos_ref, out_ref):
    out_ref[0] = x_ref[0] + pos_ref[...]


def kernel(x, pos_table):
    batch, seq_len, d_model = x.shape
    s_blocks = seq_len // _BLOCK_ROWS

    return pl.pallas_call(
        _add_kernel,
        grid=(s_blocks, batch),
        in_specs=[
            pl.BlockSpec((1, _BLOCK_ROWS, d_model), lambda s, b: (b, s, 0)),
            pl.BlockSpec((_BLOCK_ROWS, d_model), lambda s, b: (s, 0)),
        ],
        out_specs=pl.BlockSpec((1, _BLOCK_ROWS, d_model), lambda s, b: (b, s, 0)),
        out_shape=jax.ShapeDtypeStruct((batch, seq_len, d_model), x.dtype),
    )(x, pos_table)


# speedup vs baseline: 1.1822x; 1.1822x over previous
import jax
import jax.numpy as jnp
from jax.experimental import pallas as pl


_BLOCK_ROWS = 2048


def _add_kernel(x_ref, pos_ref, out_ref):
    out_ref[0] = x_ref[0] + pos_ref[...]


def kernel(x, pos_table):
    batch, seq_len, d_model = x.shape
    s_blocks = seq_len // _BLOCK_ROWS

    return pl.pallas_call(
        _add_kernel,
        grid=(s_blocks, batch),
        in_specs=[
            pl.BlockSpec((1, _BLOCK_ROWS, d_model), lambda s, b: (b, s, 0)),
            pl.BlockSpec((_BLOCK_ROWS, d_model), lambda s, b: (s, 0)),
        ],
        out_specs=pl.BlockSpec((1, _BLOCK_ROWS, d_model), lambda s, b: (b, s, 0)),
        out_shape=jax.ShapeDtypeStruct((batch, seq_len, d_model), x.dtype),
    )(x, pos_table)
